# Initial kernel scaffold; baseline (speedup 1.0000x reference)
import functools

import jax
import jax.numpy as jnp
from jax import lax
from jax.experimental import pallas as pl
from jax.experimental.pallas import tpu as pltpu
from jax.experimental.pallas import tpu_sc as plsc

B = 4096
L = 200
D = 128
LANES = 16
NGRP = D // LANES

CHUNK = 104
NCHUNK_PER_SEQ = 2
PAD = NCHUNK_PER_SEQ * CHUNK - L

NC = 2
NS = 16
NW = NC * NS
SPW = B // NW
NCH = SPW * NCHUNK_PER_SEQ


def _body(idx_hbm, bs_hbm, table_hbm, out_hbm,
          idx_v, bs_v, buf0, buf1, row0_v, out_v, sem0, sem1, sem_r):
    wid = lax.axis_index("s") * NC + lax.axis_index("c")
    seq0 = wid * SPW

    pltpu.sync_copy(idx_hbm.at[pl.ds(wid * NCH, NCH)], idx_v)
    pltpu.sync_copy(bs_hbm.at[pl.ds(seq0, SPW)], bs_v)
    pltpu.async_copy(table_hbm.at[pl.ds(0, 1)], row0_v, sem_r).wait()
    row0 = [row0_v[0, pl.ds(g * LANES, LANES)] for g in range(NGRP)]

    bufs = (buf0, buf1)
    sems = (sem0, sem1)

    pltpu.async_copy(table_hbm.at[idx_v.at[0]], buf0, sem0)
    pltpu.async_copy(table_hbm.at[idx_v.at[1]], buf1, sem1)

    def seq_body(s, carry):
        acc = tuple(jnp.zeros((LANES,), jnp.float32) for _ in range(NGRP))
        for k in range(NCHUNK_PER_SEQ):
            c = NCHUNK_PER_SEQ * s + k
            buf, sem = bufs[k], sems[k]
            pltpu.make_async_copy(table_hbm.at[idx_v.at[c]], buf, sem).wait()

            def red(r, a):
                return tuple(a[g] + buf[r, pl.ds(g * LANES, LANES)]
                             for g in range(NGRP))

            acc = lax.fori_loop(0, CHUNK, red, acc)

            @pl.when(c + NCHUNK_PER_SEQ < NCH)
            def _():
                pltpu.async_copy(
                    table_hbm.at[idx_v.at[c + NCHUNK_PER_SEQ]], buf, sem)

        bs = plsc.load_gather(bs_v, [jnp.full((LANES,), s, jnp.int32)])
        scale = 1.0 / bs.astype(jnp.float32)
        for g in range(NGRP):
            out_v[s, pl.ds(g * LANES, LANES)] = (
                acc[g] - float(PAD) * row0[g]) * scale
        return carry

    lax.fori_loop(0, SPW, seq_body, 0)
    pltpu.sync_copy(out_v, out_hbm.at[pl.ds(seq0, SPW)])


@jax.jit
def _embed_bag(idx_rows, batch_sizes, table):
    mesh = plsc.VectorSubcoreMesh(core_axis_name="c", subcore_axis_name="s")
    return pl.kernel(
        _body,
        out_type=jax.ShapeDtypeStruct((B, D), jnp.float32),
        mesh=mesh,
        scratch_types=[
            pltpu.VMEM((NCH, CHUNK), jnp.int32),
            pltpu.VMEM((SPW,), jnp.int32),
            pltpu.VMEM((CHUNK, D), jnp.float32),
            pltpu.VMEM((CHUNK, D), jnp.float32),
            pltpu.VMEM((1, D), jnp.float32),
            pltpu.VMEM((SPW, D), jnp.float32),
            pltpu.SemaphoreType.DMA,
            pltpu.SemaphoreType.DMA,
            pltpu.SemaphoreType.DMA,
        ],
    )(idx_rows, batch_sizes, table)


def kernel(word_inputs_data, batch_sizes, embedding_table):
    idx = word_inputs_data.astype(jnp.int32)
    idx = jnp.concatenate(
        [idx, jnp.zeros((B, PAD), jnp.int32)], axis=1)
    idx_rows = idx.reshape(B * NCHUNK_PER_SEQ, CHUNK)
    return _embed_bag(idx_rows, batch_sizes.astype(jnp.int32),
                      embedding_table)


# speedup vs baseline: 1.9389x; 1.9389x over previous
import functools

import jax
import jax.numpy as jnp
from jax import lax
from jax.experimental import pallas as pl
from jax.experimental.pallas import tpu as pltpu
from jax.experimental.pallas import tpu_sc as plsc

B = 4096
L = 200
D = 128
LANES = 16
NGRP = D // LANES

CHUNK = 104
NCHUNK_PER_SEQ = 2
PAD = NCHUNK_PER_SEQ * CHUNK - L

NC = 2
NS = 16
NW = NC * NS
SPW = B // NW
NCH = SPW * NCHUNK_PER_SEQ


def _body(idx_hbm, bs_hbm, table_hbm, out_hbm,
          idx_v, bs_v, buf0, buf1, row0_v, out_v, sem0, sem1, sem_r):
    wid = lax.axis_index("s") * NC + lax.axis_index("c")
    seq0 = wid * SPW

    pltpu.sync_copy(idx_hbm.at[pl.ds(wid * NCH, NCH)], idx_v)
    pltpu.sync_copy(bs_hbm.at[pl.ds(seq0, SPW)], bs_v)
    pltpu.async_copy(table_hbm.at[pl.ds(0, 1)], row0_v, sem_r).wait()
    row0 = [row0_v[0, pl.ds(g * LANES, LANES)] for g in range(NGRP)]

    bufs = (buf0, buf1)
    sems = (sem0, sem1)

    pltpu.async_copy(table_hbm.at[idx_v.at[0]], buf0, sem0)
    pltpu.async_copy(table_hbm.at[idx_v.at[1]], buf1, sem1)

    def seq_body(s, carry):
        acc = tuple(jnp.zeros((LANES,), jnp.float32) for _ in range(NGRP))
        for k in range(NCHUNK_PER_SEQ):
            c = NCHUNK_PER_SEQ * s + k
            buf, sem = bufs[k], sems[k]
            pltpu.make_async_copy(table_hbm.at[idx_v.at[c]], buf, sem).wait()

            def red(r, a):
                return tuple(a[g] + buf[r, pl.ds(g * LANES, LANES)]
                             for g in range(NGRP))

            acc = lax.fori_loop(0, CHUNK, red, acc)

            @pl.when(c + NCHUNK_PER_SEQ < NCH)
            def _():
                pltpu.async_copy(
                    table_hbm.at[idx_v.at[c + NCHUNK_PER_SEQ]], buf, sem)

        bs = plsc.load_gather(bs_v, [jnp.full((LANES,), s, jnp.int32)])
        scale = 1.0 / bs.astype(jnp.float32)
        for g in range(NGRP):
            out_v[s, pl.ds(g * LANES, LANES)] = (
                acc[g] - float(PAD) * row0[g]) * scale
        return carry

    lax.fori_loop(0, SPW, seq_body, 0)
    pltpu.sync_copy(out_v, out_hbm.at[pl.ds(seq0, SPW)])


@jax.jit
def _embed_bag(idx_rows, batch_sizes, table):
    mesh = plsc.VectorSubcoreMesh(core_axis_name="c", subcore_axis_name="s")
    return pl.kernel(
        _body,
        out_type=jax.ShapeDtypeStruct((B, D), jnp.float32),
        mesh=mesh,
        compiler_params=pltpu.CompilerParams(needs_layout_passes=False),
        scratch_types=[
            pltpu.VMEM((NCH, CHUNK), jnp.int32),
            pltpu.VMEM((SPW,), jnp.int32),
            pltpu.VMEM((CHUNK, D), jnp.float32),
            pltpu.VMEM((CHUNK, D), jnp.float32),
            pltpu.VMEM((1, D), jnp.float32),
            pltpu.VMEM((SPW, D), jnp.float32),
            pltpu.SemaphoreType.DMA,
            pltpu.SemaphoreType.DMA,
            pltpu.SemaphoreType.DMA,
        ],
    )(idx_rows, batch_sizes, table)


def kernel(word_inputs_data, batch_sizes, embedding_table):
    idx = word_inputs_data.astype(jnp.int32)
    idx = jnp.concatenate(
        [idx, jnp.zeros((B, PAD), jnp.int32)], axis=1)
    idx_rows = idx.reshape(B * NCHUNK_PER_SEQ, CHUNK)
    return _embed_bag(idx_rows, batch_sizes.astype(jnp.int32),
                      embedding_table)
